# baseline (speedup 1.0000x reference)
import dataclasses
import functools

import jax
import jax.numpy as jnp
from jax import lax
from jax.experimental import pallas as pl
from jax.experimental.pallas import tpu as pltpu
from jax.experimental.pallas import tpu_sc as plsc

N_, D_, H_, E_ = 4096, 1024, 1024, 8
BTR = 512
NBR = N_ // BTR
BT2 = 256
LOG_BT2 = 8
PAD = N_ * 2 + E_ * BT2
TB = PAD // BT2
TBP = 48
NC, NS = 2, 16
NW = NC * NS
TOK_W = N_ // NW
SLOT_W = PAD // NW
GROWS = 80
CROWS = 32


def _gelu_exact(v):
    return 0.5 * v * (1.0 + lax.erf(v * 0.7071067811865476))


def _sc_params():
    cp = pltpu.CompilerParams()
    if "needs_layout_passes" in pltpu.CompilerParams.__dataclass_fields__:
        cp = dataclasses.replace(cp, needs_layout_passes=False)
    return cp


def _a_body(xb_ref, wg_ref, bg_ref, ls_ref,
            e1_ref, e2_ref, w1_ref, w2_ref, er1_ref, er2_ref, bc_ref):
    xb = xb_ref[...]
    logits = lax.dot_general(xb, wg_ref[...], (((1,), (1,)), ((), ())),
                             preferred_element_type=jnp.float32)
    logits = logits + bg_ref[...]
    iota = lax.broadcasted_iota(jnp.int32, logits.shape, 1)
    m1 = jnp.max(logits, axis=1, keepdims=True)
    idx1 = jnp.min(jnp.where(logits == m1, iota, E_), axis=1, keepdims=True)
    oh1 = iota == idx1
    masked = jnp.where(oh1, -1e30, logits)
    m2 = jnp.max(masked, axis=1, keepdims=True)
    idx2 = jnp.min(jnp.where(masked == m2, iota, E_), axis=1, keepdims=True)
    oh2 = iota == idx2
    a = jnp.exp(m2 - m1)
    wt1 = 1.0 / (1.0 + a)
    wt2 = 1.0 - wt1
    cnt = oh1.astype(jnp.float32) + oh2.astype(jnp.float32)
    excl = lax.dot_general(ls_ref[...], cnt.astype(jnp.bfloat16),
                           (((1,), (0,)), ((), ())),
                           preferred_element_type=jnp.float32)
    er1 = jnp.sum(jnp.where(oh1, excl, 0.0), axis=1, keepdims=True)
    er2 = jnp.sum(jnp.where(oh2, excl, 0.0), axis=1, keepdims=True)
    e1_ref[0] = idx1
    e2_ref[0] = idx2
    w1_ref[0] = wt1
    w2_ref[0] = wt2
    er1_ref[0] = er1.astype(jnp.int32)
    er2_ref[0] = er2.astype(jnp.int32)
    bc_ref[0] = jnp.sum(cnt, axis=0, keepdims=True).astype(jnp.int32)


def _c1_body(e1_hbm, e2_hbm, er1_hbm, er2_hbm, bc_hbm,
             pos1_hbm, pos2_hbm, bexp_hbm,
             e1v, e2v, er1v, er2v, bcv, basev, padoffv, posv, bev):
    wid = lax.axis_index("s") * NC + lax.axis_index("c")
    base = wid * TOK_W
    pltpu.sync_copy(e1_hbm.at[pl.ds(base, TOK_W)], e1v)
    pltpu.sync_copy(e2_hbm.at[pl.ds(base, TOK_W)], e2v)
    pltpu.sync_copy(er1_hbm.at[pl.ds(base, TOK_W)], er1v)
    pltpu.sync_copy(er2_hbm.at[pl.ds(base, TOK_W)], er2v)
    pltpu.sync_copy(bc_hbm, bcv)

    iota = lax.iota(jnp.int32, 16)
    lane_ok = iota < E_
    totals = jnp.zeros((16,), jnp.int32)
    for i in range(NBR):
        totals = totals + bcv[pl.ds(16 * i, 16)]
    padded = jnp.where(lane_ok, ((totals + (BT2 - 1)) >> LOG_BT2) << LOG_BT2, 0)
    pad_off = plsc.cumsum(padded) - padded
    padoffv[...] = pad_off

    blk = wid // (BTR // TOK_W)

    def _prior(i, acc):
        return acc + bcv[pl.ds(16 * i, 16)]

    prior = lax.fori_loop(0, blk, _prior, jnp.zeros((16,), jnp.int32))
    basev[...] = pad_off + prior

    zero = jnp.zeros((16,), jnp.int32)
    for j in range(TOK_W // 16):
        sl = pl.ds(16 * j, 16)
        p1 = plsc.load_gather(basev, [e1v[sl]]) + er1v[sl]
        p2 = plsc.load_gather(basev, [e2v[sl]]) + er2v[sl]
        posv[pl.ds(16 * j, 16)] = p1
        posv[pl.ds(TOK_W + 16 * j, 16)] = p2

    pltpu.sync_copy(posv.at[pl.ds(0, TOK_W)], pos1_hbm.at[pl.ds(base, TOK_W)])
    pltpu.sync_copy(posv.at[pl.ds(TOK_W, TOK_W)], pos2_hbm.at[pl.ds(base, TOK_W)])

    @pl.when(wid == 0)
    def _():
        for jj in range(TBP // 16):
            start = (16 * jj + iota) * BT2
            cntv = jnp.zeros((16,), jnp.int32)
            for e in range(1, E_):
                off_e = plsc.load_gather(padoffv, [zero + e])
                cntv += jnp.where(start >= off_e, 1, 0)
            bev[pl.ds(16 * jj, 16)] = cntv
        pltpu.sync_copy(bev, bexp_hbm)


def _c2_body(x_hbm, pos1_hbm, pos2_hbm, w1_hbm, w2_hbm, xs_hbm, ws_hbm,
             p1v, p2v, w1v, w2v, tokv, wv, buf):
    wid = lax.axis_index("s") * NC + lax.axis_index("c")
    sbase = wid * SLOT_W
    pltpu.sync_copy(pos1_hbm, p1v)
    pltpu.sync_copy(pos2_hbm, p2v)
    pltpu.sync_copy(w1_hbm, w1v)
    pltpu.sync_copy(w2_hbm, w2v)
    iota = lax.iota(jnp.int32, 16)
    for g in range(SLOT_W // 16):
        tokv[pl.ds(16 * g, 16)] = jnp.zeros((16,), jnp.int32)
        wv[pl.ds(16 * g, 16)] = jnp.zeros((16,), jnp.float32)

    @pl.loop(0, N_ // 16)
    def _(g):
        nvec = 16 * g + iota
        sl = pl.ds(16 * g, 16)
        for pv, wv_in in ((p1v, w1v), (p2v, w2v)):
            p = pv[sl]
            rel = p - sbase
            m = (rel >= 0) & (rel < SLOT_W)
            relc = jnp.minimum(jnp.maximum(rel, 0), SLOT_W - 1)
            plsc.store_scatter(tokv, [relc], nvec, mask=m)
            plsc.store_scatter(wv, [relc], wv_in[sl], mask=m)

    pltpu.sync_copy(wv, ws_hbm.at[pl.ds(sbase, SLOT_W)])
    for c in range(SLOT_W // GROWS):
        pltpu.sync_copy(x_hbm.at[tokv.at[pl.ds(GROWS * c, GROWS)]], buf)
        pltpu.sync_copy(buf, xs_hbm.at[pl.ds(sbase + GROWS * c, GROWS)])


def _d_body(be_ref, xs_ref, w1_ref, b1_ref, w2_ref, b2_ref, ws_ref, ys_ref):
    xb = xs_ref[...].astype(jnp.bfloat16)
    h = lax.dot_general(xb, w1_ref[0], (((1,), (0,)), ((), ())),
                        preferred_element_type=jnp.float32)
    h = _gelu_exact(h + b1_ref[0])
    y = lax.dot_general(h.astype(jnp.bfloat16), w2_ref[0],
                        (((1,), (0,)), ((), ())),
                        preferred_element_type=jnp.float32)
    ys_ref[...] = (y + b2_ref[0]) * ws_ref[...]


def _e_body(ys_hbm, pos1_hbm, pos2_hbm, out_hbm, p1v, p2v, bufa, bufb):
    wid = lax.axis_index("s") * NC + lax.axis_index("c")
    base = wid * TOK_W
    pltpu.sync_copy(pos1_hbm.at[pl.ds(base, TOK_W)], p1v)
    pltpu.sync_copy(pos2_hbm.at[pl.ds(base, TOK_W)], p2v)
    for c in range(TOK_W // CROWS):
        pltpu.sync_copy(ys_hbm.at[p1v.at[pl.ds(CROWS * c, CROWS)]], bufa)
        pltpu.sync_copy(ys_hbm.at[p2v.at[pl.ds(CROWS * c, CROWS)]], bufb)

        @pl.loop(0, CROWS)
        def _(r):
            for cc in range(H_ // 16):
                bufa[r, pl.ds(16 * cc, 16)] += bufb[r, pl.ds(16 * cc, 16)]

        pltpu.sync_copy(bufa, out_hbm.at[pl.ds(base + CROWS * c, CROWS)])


def kernel(x, Wg, bg, W1, b1, W2, b2):
    B_, T_, D = x.shape
    xf = x.reshape(N_, D_)
    xb = xf.astype(jnp.bfloat16)
    wg = Wg.astype(jnp.bfloat16)
    bg2 = bg.reshape(1, E_)
    lsm = jnp.tril(jnp.ones((BTR, BTR), jnp.bfloat16), -1)
    b1r = b1.reshape(E_, 1, H_)
    b2r = b2.reshape(E_, 1, H_)
    w1t = jnp.swapaxes(W1, 1, 2).astype(jnp.bfloat16)
    w2t = jnp.swapaxes(W2, 1, 2).astype(jnp.bfloat16)

    col_i = functools.partial(jax.ShapeDtypeStruct, dtype=jnp.int32)
    col_f = functools.partial(jax.ShapeDtypeStruct, dtype=jnp.float32)
    e1o, e2o, wt1o, wt2o, er1o, er2o, bco = pl.pallas_call(
        _a_body,
        grid=(NBR,),
        in_specs=[
            pl.BlockSpec((BTR, D_), lambda i: (i, 0)),
            pl.BlockSpec((E_, D_), lambda i: (0, 0)),
            pl.BlockSpec((1, E_), lambda i: (0, 0)),
            pl.BlockSpec((BTR, BTR), lambda i: (0, 0)),
        ],
        out_specs=[
            pl.BlockSpec((1, BTR, 1), lambda i: (i, 0, 0)),
            pl.BlockSpec((1, BTR, 1), lambda i: (i, 0, 0)),
            pl.BlockSpec((1, BTR, 1), lambda i: (i, 0, 0)),
            pl.BlockSpec((1, BTR, 1), lambda i: (i, 0, 0)),
            pl.BlockSpec((1, BTR, 1), lambda i: (i, 0, 0)),
            pl.BlockSpec((1, BTR, 1), lambda i: (i, 0, 0)),
            pl.BlockSpec((1, 1, E_), lambda i: (i, 0, 0)),
        ],
        out_shape=[
            col_i((NBR, BTR, 1)), col_i((NBR, BTR, 1)),
            col_f((NBR, BTR, 1)), col_f((NBR, BTR, 1)),
            col_i((NBR, BTR, 1)), col_i((NBR, BTR, 1)),
            col_i((NBR, 1, E_)),
        ],
        compiler_params=pltpu.CompilerParams(
            dimension_semantics=("arbitrary",)),
    )(xb, wg, bg2, lsm)

    e1 = e1o.reshape(N_)
    e2 = e2o.reshape(N_)
    wt1 = wt1o.reshape(N_)
    wt2 = wt2o.reshape(N_)
    er1 = er1o.reshape(N_)
    er2 = er2o.reshape(N_)
    bcf = jnp.pad(bco.reshape(NBR, E_), ((0, 0), (0, 16 - E_))).reshape(-1)

    mesh = plsc.VectorSubcoreMesh(core_axis_name="c", subcore_axis_name="s")
    scp = _sc_params()

    c1 = pl.kernel(
        _c1_body,
        out_type=[col_i((N_,)), col_i((N_,)), col_i((TBP,))],
        mesh=mesh,
        scratch_types=[
            pltpu.VMEM((TOK_W,), jnp.int32), pltpu.VMEM((TOK_W,), jnp.int32),
            pltpu.VMEM((TOK_W,), jnp.int32), pltpu.VMEM((TOK_W,), jnp.int32),
            pltpu.VMEM((NBR * 16,), jnp.int32),
            pltpu.VMEM((16,), jnp.int32), pltpu.VMEM((16,), jnp.int32),
            pltpu.VMEM((2 * TOK_W,), jnp.int32),
            pltpu.VMEM((TBP,), jnp.int32),
        ],
        compiler_params=scp,
    )
    pos1, pos2, bexp = c1(e1, e2, er1, er2, bcf)

    c2 = pl.kernel(
        _c2_body,
        out_type=[col_f((PAD, D_)), col_f((PAD,))],
        mesh=mesh,
        scratch_types=[
            pltpu.VMEM((N_,), jnp.int32), pltpu.VMEM((N_,), jnp.int32),
            pltpu.VMEM((N_,), jnp.float32), pltpu.VMEM((N_,), jnp.float32),
            pltpu.VMEM((SLOT_W,), jnp.int32),
            pltpu.VMEM((SLOT_W,), jnp.float32),
            pltpu.VMEM((GROWS, D_), jnp.float32),
        ],
        compiler_params=scp,
    )
    xs, ws = c2(xf, pos1, pos2, wt1, wt2)

    ys = pl.pallas_call(
        _d_body,
        grid_spec=pltpu.PrefetchScalarGridSpec(
            num_scalar_prefetch=1,
            grid=(TB,),
            in_specs=[
                pl.BlockSpec((BT2, D_), lambda i, be: (i, 0)),
                pl.BlockSpec((1, D_, H_), lambda i, be: (be[i], 0, 0)),
                pl.BlockSpec((1, 1, H_), lambda i, be: (be[i], 0, 0)),
                pl.BlockSpec((1, H_, H_), lambda i, be: (be[i], 0, 0)),
                pl.BlockSpec((1, 1, H_), lambda i, be: (be[i], 0, 0)),
                pl.BlockSpec((BT2, 1), lambda i, be: (i, 0)),
            ],
            out_specs=pl.BlockSpec((BT2, H_), lambda i, be: (i, 0)),
        ),
        out_shape=jax.ShapeDtypeStruct((PAD, H_), jnp.float32),
        compiler_params=pltpu.CompilerParams(
            dimension_semantics=("arbitrary",)),
    )(bexp, xs, w1t, b1r, w2t, b2r, ws.reshape(PAD, 1))

    comb = pl.kernel(
        _e_body,
        out_type=jax.ShapeDtypeStruct((N_, H_), jnp.float32),
        mesh=mesh,
        scratch_types=[
            pltpu.VMEM((TOK_W,), jnp.int32), pltpu.VMEM((TOK_W,), jnp.int32),
            pltpu.VMEM((CROWS, H_), jnp.float32),
            pltpu.VMEM((CROWS, H_), jnp.float32),
        ],
        compiler_params=scp,
    )
    out = comb(ys, pos1, pos2)
    return out.reshape(B_, T_, H_)
